# Initial kernel scaffold; baseline (speedup 1.0000x reference)
import jax
import jax.numpy as jnp
from jax.experimental import pallas as pl


def kernel(rel_pos_hs, rel_pos_rs, rel_pos_ts, rel_neg_hs, rel_neg_rs, rel_neg_ts, rv_ent_embeds, rel_embeds):
    raise NotImplementedError("write your pallas kernel here")


# speedup vs baseline: 1.2727x; 1.2727x over previous
import functools

import jax
import jax.numpy as jnp
from jax import lax
from jax.experimental import pallas as pl
from jax.experimental.pallas import tpu as pltpu
from jax.experimental.pallas import tpu_sc as plsc

NUM_ENT = 100000
NUM_REL = 1000
D = 128
B = 16384
L = 16
CHUNK = 128


_GATHER_DNUMS = lax.GatherDimensionNumbers(
    offset_dims=(), collapsed_slice_dims=(0,), start_index_map=(0,))


def _shuffle(v, idx):
    return lax.gather(v, idx[:, None], _GATHER_DNUMS, slice_sizes=(1,),
                      mode=lax.GatherScatterMode.PROMISE_IN_BOUNDS)


def _normalize_chunk(rows_v, nrows, perms):
    def row_body(r, carry):
        vs = []
        acc = None
        for c in range(D // L):
            v = rows_v[r, pl.ds(c * L, L)]
            vs.append(v)
            sq = v * v
            acc = sq if acc is None else acc + sq
        for p in perms:
            acc = acc + _shuffle(acc, p)
        sv = jnp.maximum(acc, 1e-24)
        i = lax.bitcast_convert_type(sv, jnp.int32)
        i = 0x5F3759DF - (i >> 1)
        y = lax.bitcast_convert_type(i, jnp.float32)
        for _ in range(3):
            y = y * (1.5 - 0.5 * sv * y * y)
        for c in range(D // L):
            rows_v[r, pl.ds(c * L, L)] = vs[c] * y
        return carry

    lax.fori_loop(0, nrows, row_body, 0)


def _sc_lookup(ph, pr, pt, nh, nr, nt, ent, rel):
    try:
        info = plsc.get_sparse_core_info()
        nc, ns = info.num_cores, info.num_subcores
    except Exception:
        nc, ns = 2, 16
    nw = nc * ns
    b_per_w = B // nw
    n_chunks = b_per_w // CHUNK
    idx_rows_per_w = b_per_w // 128

    mesh = plsc.VectorSubcoreMesh(core_axis_name="c", subcore_axis_name="s",
                                  num_cores=nc, num_subcores=ns)
    out_t = [jax.ShapeDtypeStruct((B, D), jnp.float32)] * 6

    @functools.partial(
        pl.kernel,
        out_type=out_t,
        mesh=mesh,
        scratch_types=[
            pltpu.VMEM((idx_rows_per_w, 128), jnp.int32),
            pltpu.VMEM((CHUNK, D), jnp.float32),
            pltpu.SemaphoreType.DMA,
        ],
    )
    def k(ph_h, pr_h, pt_h, nh_h, nr_h, nt_h, ent_h, rel_h,
          o0, o1, o2, o3, o4, o5, idx_v, rows_v, sem):
        wid = lax.axis_index("s") * nc + lax.axis_index("c")
        base = wid * b_per_w
        lanes = lax.iota(jnp.int32, L)
        perms = [lanes ^ k for k in (8, 4, 2, 1)]
        triples = (
            (ph_h, ent_h, o0),
            (pr_h, rel_h, o1),
            (pt_h, ent_h, o2),
            (nh_h, ent_h, o3),
            (nr_h, rel_h, o4),
            (nt_h, ent_h, o5),
        )
        for idx_h, tab_h, out_h in triples:
            pltpu.sync_copy(idx_h.at[pl.ds(wid * idx_rows_per_w,
                                           idx_rows_per_w)], idx_v)
            for j in range(n_chunks):
                pltpu.async_copy(tab_h.at[idx_v.at[j]], rows_v, sem).wait()
                _normalize_chunk(rows_v, CHUNK, perms)
                pltpu.sync_copy(rows_v,
                                out_h.at[pl.ds(base + j * CHUNK, CHUNK)])

    return tuple(k(ph, pr, pt, nh, nr, nt, ent, rel))


def kernel(rel_pos_hs, rel_pos_rs, rel_pos_ts,
           rel_neg_hs, rel_neg_rs, rel_neg_ts,
           rv_ent_embeds, rel_embeds):
    def prep(ix):
        return ix.astype(jnp.int32).reshape(B // 128, 128)

    return _sc_lookup(prep(rel_pos_hs), prep(rel_pos_rs), prep(rel_pos_ts),
                      prep(rel_neg_hs), prep(rel_neg_rs), prep(rel_neg_ts),
                      rv_ent_embeds, rel_embeds)


# speedup vs baseline: 1.7264x; 1.3565x over previous
import functools

import jax
import jax.numpy as jnp
from jax import lax
from jax.experimental import pallas as pl
from jax.experimental.pallas import tpu as pltpu
from jax.experimental.pallas import tpu_sc as plsc

NUM_ENT = 100000
NUM_REL = 1000
D = 128
B = 16384
L = 16
CHUNK = 128
NBUF = 4

_GATHER_DNUMS = lax.GatherDimensionNumbers(
    offset_dims=(), collapsed_slice_dims=(0,), start_index_map=(0,))


def _shuffle(v, idx):
    return lax.gather(v, idx[:, None], _GATHER_DNUMS, slice_sizes=(1,),
                      mode=lax.GatherScatterMode.PROMISE_IN_BOUNDS)


def _normalize_chunk(rows_v, nrows, perms):
    def row_body(r, carry):
        vs = []
        acc = None
        for c in range(D // L):
            v = rows_v[r, pl.ds(c * L, L)]
            vs.append(v)
            sq = v * v
            acc = sq if acc is None else acc + sq
        for p in perms:
            acc = acc + _shuffle(acc, p)
        sv = jnp.maximum(acc, 1e-24)
        i = lax.bitcast_convert_type(sv, jnp.int32)
        i = 0x5F3759DF - (i >> 1)
        y = lax.bitcast_convert_type(i, jnp.float32)
        for _ in range(3):
            y = y * (1.5 - 0.5 * sv * y * y)
        for c in range(D // L):
            rows_v[r, pl.ds(c * L, L)] = vs[c] * y
        return carry

    lax.fori_loop(0, nrows, row_body, 0)


def _sc_lookup(idx_all, ent, rel):
    try:
        info = plsc.get_sparse_core_info()
        nc, ns = info.num_cores, info.num_subcores
    except Exception:
        nc, ns = 2, 16
    nw = nc * ns
    b_per_w = B // nw
    n_chunks = b_per_w // CHUNK
    n_total = 6 * n_chunks

    mesh = plsc.VectorSubcoreMesh(core_axis_name="c", subcore_axis_name="s",
                                  num_cores=nc, num_subcores=ns)
    out_t = [jax.ShapeDtypeStruct((B, D), jnp.float32)] * 6

    @functools.partial(
        pl.kernel,
        out_type=out_t,
        mesh=mesh,
        scratch_types=[
            pltpu.VMEM((n_total, CHUNK), jnp.int32),
            pltpu.VMEM((NBUF, CHUNK, D), jnp.float32),
        ] + [pltpu.SemaphoreType.DMA] * (2 * NBUF),
    )
    def k(idx_h, ent_h, rel_h, o0, o1, o2, o3, o4, o5, idx_v, rows_v, *sems):
        gsem = sems[:NBUF]
        ssem = sems[NBUF:]
        wid = lax.axis_index("s") * nc + lax.axis_index("c")
        base = wid * b_per_w
        lanes = lax.iota(jnp.int32, L)
        perms = [lanes ^ kk for kk in (8, 4, 2, 1)]

        pltpu.sync_copy(idx_h.at[wid], idx_v)

        tabs = (ent_h, rel_h, ent_h, ent_h, rel_h, ent_h)
        outs = (o0, o1, o2, o3, o4, o5)
        chunks = [(tabs[t], outs[t], t * n_chunks + j, j * CHUNK)
                  for t in range(6) for j in range(n_chunks)]

        g_desc = [None] * n_total
        s_desc = [None] * n_total

        def start_gather(c):
            tab, _, irow, _ = chunks[c]
            b = c % NBUF
            g_desc[c] = pltpu.async_copy(tab.at[idx_v.at[irow]],
                                         rows_v.at[b], gsem[b])

        start_gather(0)
        start_gather(1)
        for c in range(n_total):
            b = c % NBUF
            _, out_h, _, ooff = chunks[c]
            g_desc[c].wait()
            if c + 2 < n_total:
                if c + 2 - NBUF >= 0:
                    s_desc[c + 2 - NBUF].wait()
                start_gather(c + 2)
            _normalize_chunk(rows_v.at[b], CHUNK, perms)
            s_desc[c] = pltpu.async_copy(rows_v.at[b],
                                         out_h.at[pl.ds(base + ooff, CHUNK)],
                                         ssem[b])
        for c in range(n_total - NBUF, n_total):
            if s_desc[c] is not None:
                s_desc[c].wait()

    return tuple(k(idx_all, ent, rel))


def kernel(rel_pos_hs, rel_pos_rs, rel_pos_ts,
           rel_neg_hs, rel_neg_rs, rel_neg_ts,
           rv_ent_embeds, rel_embeds):
    nw = 32
    idx_all = jnp.stack(
        [ix.astype(jnp.int32).reshape(nw, B // nw // CHUNK, CHUNK)
         for ix in (rel_pos_hs, rel_pos_rs, rel_pos_ts,
                    rel_neg_hs, rel_neg_rs, rel_neg_ts)],
        axis=1,
    ).reshape(nw, 6 * (B // nw // CHUNK), CHUNK)
    return _sc_lookup(idx_all, rv_ent_embeds, rel_embeds)


# speedup vs baseline: 2.9829x; 1.7278x over previous
import functools

import jax
import jax.numpy as jnp
from jax import lax
from jax.experimental import pallas as pl
from jax.experimental.pallas import tpu as pltpu
from jax.experimental.pallas import tpu_sc as plsc

NUM_ENT = 100000
NUM_REL = 1000
D = 128
B = 16384
L = 16
CHUNK = 128
NBUF = 4

_GATHER_DNUMS = lax.GatherDimensionNumbers(
    offset_dims=(), collapsed_slice_dims=(0,), start_index_map=(0,))


def _shuffle(v, idx):
    return lax.gather(v, idx[:, None], _GATHER_DNUMS, slice_sizes=(1,),
                      mode=lax.GatherScatterMode.PROMISE_IN_BOUNDS)


def _normalize_chunk(rows_v, nrows, perms):
    def row_body(r, carry):
        vs = []
        acc = None
        for c in range(D // L):
            v = rows_v[r, pl.ds(c * L, L)]
            vs.append(v)
            sq = v * v
            acc = sq if acc is None else acc + sq
        for p in perms:
            acc = acc + _shuffle(acc, p)
        sv = jnp.maximum(acc, 1e-24)
        i = lax.bitcast_convert_type(sv, jnp.int32)
        i = 0x5F3759DF - (i >> 1)
        y = lax.bitcast_convert_type(i, jnp.float32)
        for _ in range(2):
            y = y * (1.5 - 0.5 * sv * y * y)
        for c in range(D // L):
            rows_v[r, pl.ds(c * L, L)] = vs[c] * y
        return carry

    lax.fori_loop(0, nrows, row_body, 0, unroll=2)


def _sc_lookup(idx_all, ent, rel):
    try:
        info = plsc.get_sparse_core_info()
        nc, ns = info.num_cores, info.num_subcores
    except Exception:
        nc, ns = 2, 16
    nw = nc * ns
    b_per_w = B // nw
    n_chunks = b_per_w // CHUNK
    n_total = 6 * n_chunks

    mesh = plsc.VectorSubcoreMesh(core_axis_name="c", subcore_axis_name="s",
                                  num_cores=nc, num_subcores=ns)
    out_t = [jax.ShapeDtypeStruct((B, D), jnp.float32)] * 6

    @functools.partial(
        pl.kernel,
        out_type=out_t,
        mesh=mesh,
        scratch_types=[
            pltpu.VMEM((n_total, CHUNK), jnp.int32),
            pltpu.VMEM((NBUF, CHUNK, D), jnp.float32),
        ] + [pltpu.SemaphoreType.DMA] * (2 * NBUF),
    )
    def k(idx_h, ent_h, rel_h, o0, o1, o2, o3, o4, o5, idx_v, rows_v, *sems):
        gsem = sems[:NBUF]
        ssem = sems[NBUF:]
        wid = lax.axis_index("s") * nc + lax.axis_index("c")
        base = wid * b_per_w
        lanes = lax.iota(jnp.int32, L)
        perms = [lanes ^ kk for kk in (8, 4, 2, 1)]

        pltpu.sync_copy(idx_h.at[wid], idx_v)

        tabs = (ent_h, rel_h, ent_h, ent_h, rel_h, ent_h)
        outs = (o0, o1, o2, o3, o4, o5)
        chunks = [(tabs[t], outs[t], t * n_chunks + j, j * CHUNK)
                  for t in range(6) for j in range(n_chunks)]

        g_desc = [None] * n_total
        s_desc = [None] * n_total

        def start_gather(c):
            tab, _, irow, _ = chunks[c]
            b = c % NBUF
            g_desc[c] = pltpu.async_copy(tab.at[idx_v.at[irow]],
                                         rows_v.at[b], gsem[b])

        start_gather(0)
        start_gather(1)
        for c in range(n_total):
            b = c % NBUF
            _, out_h, _, ooff = chunks[c]
            g_desc[c].wait()
            if c + 2 < n_total:
                if c + 2 - NBUF >= 0:
                    s_desc[c + 2 - NBUF].wait()
                start_gather(c + 2)
            _normalize_chunk(rows_v.at[b], CHUNK, perms)
            s_desc[c] = pltpu.async_copy(rows_v.at[b],
                                         out_h.at[pl.ds(base + ooff, CHUNK)],
                                         ssem[b])
        for c in range(n_total - NBUF, n_total):
            if s_desc[c] is not None:
                s_desc[c].wait()

    return tuple(k(idx_all, ent, rel))


def kernel(rel_pos_hs, rel_pos_rs, rel_pos_ts,
           rel_neg_hs, rel_neg_rs, rel_neg_ts,
           rv_ent_embeds, rel_embeds):
    nw = 32
    idx_all = jnp.stack(
        [ix.astype(jnp.int32).reshape(nw, B // nw // CHUNK, CHUNK)
         for ix in (rel_pos_hs, rel_pos_rs, rel_pos_ts,
                    rel_neg_hs, rel_neg_rs, rel_neg_ts)],
        axis=1,
    ).reshape(nw, 6 * (B // nw // CHUNK), CHUNK)
    return _sc_lookup(idx_all, rv_ent_embeds, rel_embeds)


# speedup vs baseline: 3.5029x; 1.1743x over previous
import functools

import jax
import jax.numpy as jnp
from jax import lax
from jax.experimental import pallas as pl
from jax.experimental.pallas import tpu as pltpu
from jax.experimental.pallas import tpu_sc as plsc

NUM_ENT = 100000
NUM_REL = 1000
D = 128
B = 16384
L = 16
CHUNK = 128
NBUF = 4

_GATHER_DNUMS = lax.GatherDimensionNumbers(
    offset_dims=(), collapsed_slice_dims=(0,), start_index_map=(0,))


def _shuffle(v, idx):
    return lax.gather(v, idx[:, None], _GATHER_DNUMS, slice_sizes=(1,),
                      mode=lax.GatherScatterMode.PROMISE_IN_BOUNDS)


def _normalize_chunk(rows_v, nrows, perms):
    def row_body(r, carry):
        vs = []
        acc = None
        for c in range(D // L):
            v = rows_v[r, pl.ds(c * L, L)]
            vs.append(v)
            sq = v * v
            acc = sq if acc is None else acc + sq
        for p in perms:
            acc = acc + _shuffle(acc, p)
        sv = jnp.maximum(acc, 1e-24)
        i = lax.bitcast_convert_type(sv, jnp.int32)
        i = 0x5F3759DF - (i >> 1)
        y = lax.bitcast_convert_type(i, jnp.float32)
        for _ in range(2):
            y = y * (1.5 - 0.5 * sv * y * y)
        for c in range(D // L):
            rows_v[r, pl.ds(c * L, L)] = vs[c] * y
        return carry

    lax.fori_loop(0, nrows, row_body, 0, unroll=4)


def _sc_lookup(idx_all, ent, rel):
    try:
        info = plsc.get_sparse_core_info()
        nc, ns = info.num_cores, info.num_subcores
    except Exception:
        nc, ns = 2, 16
    nw = nc * ns
    b_per_w = B // nw
    n_chunks = b_per_w // CHUNK
    n_total = 6 * n_chunks

    mesh = plsc.VectorSubcoreMesh(core_axis_name="c", subcore_axis_name="s",
                                  num_cores=nc, num_subcores=ns)
    out_t = [jax.ShapeDtypeStruct((B, D), jnp.float32)] * 6

    @functools.partial(
        pl.kernel,
        out_type=out_t,
        mesh=mesh,
        scratch_types=[
            pltpu.VMEM((n_total, CHUNK), jnp.int32),
            pltpu.VMEM((NBUF, CHUNK, D), jnp.float32),
        ] + [pltpu.SemaphoreType.DMA] * (2 * NBUF),
    )
    def k(idx_h, ent_h, rel_h, o0, o1, o2, o3, o4, o5, idx_v, rows_v, *sems):
        gsem = sems[:NBUF]
        ssem = sems[NBUF:]
        wid = lax.axis_index("s") * nc + lax.axis_index("c")
        base = wid * b_per_w
        lanes = lax.iota(jnp.int32, L)
        perms = [lanes ^ kk for kk in (8, 4, 2, 1)]

        pltpu.sync_copy(idx_h.at[wid], idx_v)

        tabs = (ent_h, rel_h, ent_h, ent_h, rel_h, ent_h)
        outs = (o0, o1, o2, o3, o4, o5)
        chunks = [(tabs[t], outs[t], t * n_chunks + j, j * CHUNK)
                  for t in range(6) for j in range(n_chunks)]

        g_desc = [None] * n_total
        s_desc = [None] * n_total

        def start_gather(c):
            tab, _, irow, _ = chunks[c]
            b = c % NBUF
            g_desc[c] = pltpu.async_copy(tab.at[idx_v.at[irow]],
                                         rows_v.at[b], gsem[b])

        start_gather(0)
        start_gather(1)
        for c in range(n_total):
            b = c % NBUF
            _, out_h, _, ooff = chunks[c]
            g_desc[c].wait()
            if c + 2 < n_total:
                if c + 2 - NBUF >= 0:
                    s_desc[c + 2 - NBUF].wait()
                start_gather(c + 2)
            _normalize_chunk(rows_v.at[b], CHUNK, perms)
            s_desc[c] = pltpu.async_copy(rows_v.at[b],
                                         out_h.at[pl.ds(base + ooff, CHUNK)],
                                         ssem[b])
        for c in range(n_total - NBUF, n_total):
            if s_desc[c] is not None:
                s_desc[c].wait()

    return tuple(k(idx_all, ent, rel))


def kernel(rel_pos_hs, rel_pos_rs, rel_pos_ts,
           rel_neg_hs, rel_neg_rs, rel_neg_ts,
           rv_ent_embeds, rel_embeds):
    nw = 32
    idx_all = jnp.stack(
        [ix.astype(jnp.int32).reshape(nw, B // nw // CHUNK, CHUNK)
         for ix in (rel_pos_hs, rel_pos_rs, rel_pos_ts,
                    rel_neg_hs, rel_neg_rs, rel_neg_ts)],
        axis=1,
    ).reshape(nw, 6 * (B // nw // CHUNK), CHUNK)
    return _sc_lookup(idx_all, rv_ent_embeds, rel_embeds)


# speedup vs baseline: 3.5158x; 1.0037x over previous
import functools

import jax
import jax.numpy as jnp
from jax import lax
from jax.experimental import pallas as pl
from jax.experimental.pallas import tpu as pltpu
from jax.experimental.pallas import tpu_sc as plsc

NUM_ENT = 100000
NUM_REL = 1000
D = 128
B = 16384
L = 16
CHUNK = 128
NBUF = 6
PREF = 3

_GATHER_DNUMS = lax.GatherDimensionNumbers(
    offset_dims=(), collapsed_slice_dims=(0,), start_index_map=(0,))


def _shuffle(v, idx):
    return lax.gather(v, idx[:, None], _GATHER_DNUMS, slice_sizes=(1,),
                      mode=lax.GatherScatterMode.PROMISE_IN_BOUNDS)


def _normalize_chunk(rows_v, nrows, perms):
    def row_body(r, carry):
        vs = []
        acc = None
        for c in range(D // L):
            v = rows_v[r, pl.ds(c * L, L)]
            vs.append(v)
            sq = v * v
            acc = sq if acc is None else acc + sq
        for p in perms:
            acc = acc + _shuffle(acc, p)
        sv = jnp.maximum(acc, 1e-24)
        i = lax.bitcast_convert_type(sv, jnp.int32)
        i = 0x5F3759DF - (i >> 1)
        y = lax.bitcast_convert_type(i, jnp.float32)
        for _ in range(2):
            y = y * (1.5 - 0.5 * sv * y * y)
        for c in range(D // L):
            rows_v[r, pl.ds(c * L, L)] = vs[c] * y
        return carry

    lax.fori_loop(0, nrows, row_body, 0, unroll=4)


def _sc_lookup(iph, ipr, ipt, inh, inr, int_, ent, rel):
    try:
        info = plsc.get_sparse_core_info()
        nc, ns = info.num_cores, info.num_subcores
    except Exception:
        nc, ns = 2, 16
    nw = nc * ns
    b_per_w = B // nw
    n_chunks = b_per_w // CHUNK
    n_total = 6 * n_chunks

    mesh = plsc.VectorSubcoreMesh(core_axis_name="c", subcore_axis_name="s",
                                  num_cores=nc, num_subcores=ns)
    out_t = [jax.ShapeDtypeStruct((B, D), jnp.float32)] * 6

    @functools.partial(
        pl.kernel,
        out_type=out_t,
        mesh=mesh,
        scratch_types=[
            pltpu.VMEM((n_total, CHUNK), jnp.int32),
            pltpu.VMEM((NBUF, CHUNK, D), jnp.float32),
        ] + [pltpu.SemaphoreType.DMA] * (2 * NBUF + 1),
    )
    def k(iph_h, ipr_h, ipt_h, inh_h, inr_h, int_h, ent_h, rel_h,
          o0, o1, o2, o3, o4, o5, idx_v, rows_v, *sems):
        gsem = sems[:NBUF]
        ssem = sems[NBUF:2 * NBUF]
        isem = sems[2 * NBUF]
        wid = lax.axis_index("s") * nc + lax.axis_index("c")
        base = wid * b_per_w
        lanes = lax.iota(jnp.int32, L)
        perms = [lanes ^ kk for kk in (8, 4, 2, 1)]

        idxs = (iph_h, ipr_h, ipt_h, inh_h, inr_h, int_h)
        idesc = [
            pltpu.async_copy(
                idxs[t].at[pl.ds(wid * n_chunks, n_chunks)],
                idx_v.at[pl.ds(t * n_chunks, n_chunks)], isem)
            for t in range(6)
        ]
        for dsc in idesc:
            dsc.wait()

        tabs = (ent_h, rel_h, ent_h, ent_h, rel_h, ent_h)
        outs = (o0, o1, o2, o3, o4, o5)
        chunks = [(tabs[t], outs[t], t * n_chunks + j, j * CHUNK)
                  for t in range(6) for j in range(n_chunks)]

        g_desc = [None] * n_total
        s_desc = [None] * n_total

        def start_gather(c):
            tab, _, irow, _ = chunks[c]
            b = c % NBUF
            g_desc[c] = pltpu.async_copy(tab.at[idx_v.at[irow]],
                                         rows_v.at[b], gsem[b])

        for c in range(PREF):
            start_gather(c)
        for c in range(n_total):
            b = c % NBUF
            _, out_h, _, ooff = chunks[c]
            g_desc[c].wait()
            if c + PREF < n_total:
                if c + PREF - NBUF >= 0:
                    s_desc[c + PREF - NBUF].wait()
                start_gather(c + PREF)
            _normalize_chunk(rows_v.at[b], CHUNK, perms)
            s_desc[c] = pltpu.async_copy(rows_v.at[b],
                                         out_h.at[pl.ds(base + ooff, CHUNK)],
                                         ssem[b])
        for c in range(n_total - NBUF, n_total):
            if s_desc[c] is not None:
                s_desc[c].wait()

    return tuple(k(iph, ipr, ipt, inh, inr, int_, ent, rel))


def kernel(rel_pos_hs, rel_pos_rs, rel_pos_ts,
           rel_neg_hs, rel_neg_rs, rel_neg_ts,
           rv_ent_embeds, rel_embeds):
    def prep(ix):
        return ix.astype(jnp.int32).reshape(B // CHUNK, CHUNK)

    return _sc_lookup(prep(rel_pos_hs), prep(rel_pos_rs), prep(rel_pos_ts),
                      prep(rel_neg_hs), prep(rel_neg_rs), prep(rel_neg_ts),
                      rv_ent_embeds, rel_embeds)


# speedup vs baseline: 3.9430x; 1.1215x over previous
import functools

import jax
import jax.numpy as jnp
from jax import lax
from jax.experimental import pallas as pl
from jax.experimental.pallas import tpu as pltpu
from jax.experimental.pallas import tpu_sc as plsc

NUM_ENT = 100000
NUM_REL = 1000
D = 128
B = 16384
L = 16
CHUNK = 128
NBUF = 6
PREF = 3
REL_W = 8
REL_ROWS = 128
REL_LAST = NUM_REL - (REL_W - 1) * REL_ROWS

_GATHER_DNUMS = lax.GatherDimensionNumbers(
    offset_dims=(), collapsed_slice_dims=(0,), start_index_map=(0,))


def _shuffle(v, idx):
    return lax.gather(v, idx[:, None], _GATHER_DNUMS, slice_sizes=(1,),
                      mode=lax.GatherScatterMode.PROMISE_IN_BOUNDS)


def _normalize_chunk(rows_v, nrows, perms, unroll=4):
    def row_body(r, carry):
        vs = []
        acc = None
        for c in range(D // L):
            v = rows_v[r, pl.ds(c * L, L)]
            vs.append(v)
            sq = v * v
            acc = sq if acc is None else acc + sq
        for p in perms:
            acc = acc + _shuffle(acc, p)
        sv = jnp.maximum(acc, 1e-24)
        i = lax.bitcast_convert_type(sv, jnp.int32)
        i = 0x5F3759DF - (i >> 1)
        y = lax.bitcast_convert_type(i, jnp.float32)
        for _ in range(2):
            y = y * (1.5 - 0.5 * sv * y * y)
        for c in range(D // L):
            rows_v[r, pl.ds(c * L, L)] = vs[c] * y
        return carry

    lax.fori_loop(0, nrows, row_body, 0, unroll=unroll)


def _sc_lookup(iph, ipr, ipt, inh, inr, int_, ent, rel):
    try:
        info = plsc.get_sparse_core_info()
        nc, ns = info.num_cores, info.num_subcores
    except Exception:
        nc, ns = 2, 16
    nw = nc * ns
    b_per_w = B // nw
    n_chunks = b_per_w // CHUNK
    n_total = 6 * n_chunks

    mesh = plsc.VectorSubcoreMesh(core_axis_name="c", subcore_axis_name="s",
                                  num_cores=nc, num_subcores=ns)
    out_t = [jax.ShapeDtypeStruct((B, D), jnp.float32)] * 6

    @functools.partial(
        pl.kernel,
        out_type=out_t,
        mesh=mesh,
        scratch_types=[
            pltpu.VMEM((n_total, CHUNK), jnp.int32),
            pltpu.VMEM((NBUF, CHUNK, D), jnp.float32),
            pltpu.VMEM_SHARED((NUM_REL, D), jnp.float32),
        ] + [pltpu.SemaphoreType.DMA] * (2 * NBUF + 1),
    )
    def k(iph_h, ipr_h, ipt_h, inh_h, inr_h, int_h, ent_h, rel_h,
          o0, o1, o2, o3, o4, o5, idx_v, rows_v, rel_sp, *sems):
        gsem = sems[:NBUF]
        ssem = sems[NBUF:2 * NBUF]
        isem = sems[2 * NBUF]
        sid = lax.axis_index("s")
        wid = sid * nc + lax.axis_index("c")
        base = wid * b_per_w
        lanes = lax.iota(jnp.int32, L)
        perms = [lanes ^ kk for kk in (8, 4, 2, 1)]

        idxs = (iph_h, ipr_h, ipt_h, inh_h, inr_h, int_h)
        idesc = [
            pltpu.async_copy(
                idxs[t].at[pl.ds(wid * n_chunks, n_chunks)],
                idx_v.at[pl.ds(t * n_chunks, n_chunks)], isem)
            for t in range(6)
        ]

        def _prep_rel(nrows):
            r0 = sid * REL_ROWS
            pltpu.sync_copy(rel_h.at[pl.ds(r0, nrows)],
                            rows_v.at[0, pl.ds(0, nrows)])
            _normalize_chunk(rows_v.at[0], nrows, perms)
            pltpu.sync_copy(rows_v.at[0, pl.ds(0, nrows)],
                            rel_sp.at[pl.ds(r0, nrows)])

        @pl.when(sid < REL_W - 1)
        def _prep_full():
            _prep_rel(REL_ROWS)

        @pl.when(sid == REL_W - 1)
        def _prep_tail():
            _prep_rel(REL_LAST)

        plsc.subcore_barrier()
        for dsc in idesc:
            dsc.wait()

        tabs = (ent_h, rel_sp, ent_h, ent_h, rel_sp, ent_h)
        norm = (True, False, True, True, False, True)
        outs = (o0, o1, o2, o3, o4, o5)
        chunks = [(tabs[t], outs[t], t * n_chunks + j, j * CHUNK, norm[t])
                  for t in range(6) for j in range(n_chunks)]

        g_desc = [None] * n_total
        s_desc = [None] * n_total

        def start_gather(c):
            tab, _, irow, _, _ = chunks[c]
            b = c % NBUF
            g_desc[c] = pltpu.async_copy(tab.at[idx_v.at[irow]],
                                         rows_v.at[b], gsem[b])

        for c in range(PREF):
            start_gather(c)
        for c in range(n_total):
            b = c % NBUF
            _, out_h, _, ooff, do_norm = chunks[c]
            g_desc[c].wait()
            if c + PREF < n_total:
                if c + PREF - NBUF >= 0:
                    s_desc[c + PREF - NBUF].wait()
                start_gather(c + PREF)
            if do_norm:
                _normalize_chunk(rows_v.at[b], CHUNK, perms)
            s_desc[c] = pltpu.async_copy(rows_v.at[b],
                                         out_h.at[pl.ds(base + ooff, CHUNK)],
                                         ssem[b])
        for c in range(n_total - NBUF, n_total):
            if s_desc[c] is not None:
                s_desc[c].wait()

    return tuple(k(iph, ipr, ipt, inh, inr, int_, ent, rel))


def kernel(rel_pos_hs, rel_pos_rs, rel_pos_ts,
           rel_neg_hs, rel_neg_rs, rel_neg_ts,
           rv_ent_embeds, rel_embeds):
    def prep(ix):
        return ix.astype(jnp.int32).reshape(B // CHUNK, CHUNK)

    return _sc_lookup(prep(rel_pos_hs), prep(rel_pos_rs), prep(rel_pos_ts),
                      prep(rel_neg_hs), prep(rel_neg_rs), prep(rel_neg_ts),
                      rv_ent_embeds, rel_embeds)


# speedup vs baseline: 3.9833x; 1.0102x over previous
import functools

import jax
import jax.numpy as jnp
from jax import lax
from jax.experimental import pallas as pl
from jax.experimental.pallas import tpu as pltpu
from jax.experimental.pallas import tpu_sc as plsc

NUM_ENT = 100000
NUM_REL = 1000
D = 128
B = 16384
L = 16
CHUNK = 128
NBUF = 7
PREF = 4
REL_W = 8
REL_PAD = REL_W * CHUNK
REL_ROWS = REL_PAD // REL_W

_GATHER_DNUMS = lax.GatherDimensionNumbers(
    offset_dims=(), collapsed_slice_dims=(0,), start_index_map=(0,))


def _shuffle(v, idx):
    return lax.gather(v, idx[:, None], _GATHER_DNUMS, slice_sizes=(1,),
                      mode=lax.GatherScatterMode.PROMISE_IN_BOUNDS)


def _normalize_chunk(rows_v, nrows, perms, unroll=4):
    def row_body(r, carry):
        vs = []
        acc = None
        for c in range(D // L):
            v = rows_v[r, pl.ds(c * L, L)]
            vs.append(v)
            sq = v * v
            acc = sq if acc is None else acc + sq
        for p in perms:
            acc = acc + _shuffle(acc, p)
        sv = jnp.maximum(acc, 1e-24)
        i = lax.bitcast_convert_type(sv, jnp.int32)
        i = 0x5F3759DF - (i >> 1)
        y = lax.bitcast_convert_type(i, jnp.float32)
        for _ in range(2):
            y = y * (1.5 - 0.5 * sv * y * y)
        for c in range(D // L):
            rows_v[r, pl.ds(c * L, L)] = vs[c] * y
        return carry

    lax.fori_loop(0, nrows, row_body, 0, unroll=unroll)


def _sc_lookup(iph, ipr, ipt, inh, inr, int_, ent, rel):
    try:
        info = plsc.get_sparse_core_info()
        nc, ns = info.num_cores, info.num_subcores
    except Exception:
        nc, ns = 2, 16
    nw = nc * ns
    b_per_w = B // nw
    n_chunks = b_per_w // CHUNK
    n_total = 6 * n_chunks

    mesh = plsc.VectorSubcoreMesh(core_axis_name="c", subcore_axis_name="s",
                                  num_cores=nc, num_subcores=ns)
    out_t = [jax.ShapeDtypeStruct((B, D), jnp.float32)] * 6

    @functools.partial(
        pl.kernel,
        out_type=out_t,
        mesh=mesh,
        scratch_types=[
            pltpu.VMEM((n_total, CHUNK), jnp.int32),
            pltpu.VMEM((NBUF, CHUNK, D), jnp.float32),
            pltpu.VMEM_SHARED((REL_PAD, D), jnp.float32),
        ] + [pltpu.SemaphoreType.DMA] * (2 * NBUF + 1),
    )
    def k(iph_h, ipr_h, ipt_h, inh_h, inr_h, int_h, ent_h, rel_h,
          o0, o1, o2, o3, o4, o5, idx_v, rows_v, rel_sp, *sems):
        gsem = sems[:NBUF]
        ssem = sems[NBUF:2 * NBUF]
        isem = sems[2 * NBUF]
        sid = lax.axis_index("s")
        wid = sid * nc + lax.axis_index("c")
        base = wid * b_per_w
        lanes = lax.iota(jnp.int32, L)
        perms = [lanes ^ kk for kk in (8, 4, 2, 1)]

        idxs = (iph_h, ipr_h, ipt_h, inh_h, inr_h, int_h)
        idesc = [
            pltpu.async_copy(
                idxs[t].at[pl.ds(wid * n_chunks, n_chunks)],
                idx_v.at[pl.ds(t * n_chunks, n_chunks)], isem)
            for t in range(6)
        ]

        @pl.when(sid < REL_W)
        def _prep_rel():
            r0 = sid * REL_ROWS
            pltpu.sync_copy(rel_h.at[pl.ds(r0, REL_ROWS)],
                            rows_v.at[0, pl.ds(0, REL_ROWS)])
            _normalize_chunk(rows_v.at[0], REL_ROWS, perms)
            pltpu.sync_copy(rows_v.at[0, pl.ds(0, REL_ROWS)],
                            rel_sp.at[pl.ds(r0, REL_ROWS)])

        plsc.subcore_barrier()
        for dsc in idesc:
            dsc.wait()

        tabs = (ent_h, rel_sp, ent_h, ent_h, rel_sp, ent_h)
        norm = (True, False, True, True, False, True)
        outs = (o0, o1, o2, o3, o4, o5)
        chunks = [(tabs[t], outs[t], t * n_chunks + j, j * CHUNK, norm[t])
                  for t in range(6) for j in range(n_chunks)]

        g_desc = [None] * n_total
        s_desc = [None] * n_total

        def start_gather(c):
            tab, _, irow, _, _ = chunks[c]
            b = c % NBUF
            g_desc[c] = pltpu.async_copy(tab.at[idx_v.at[irow]],
                                         rows_v.at[b], gsem[b])

        for c in range(PREF):
            start_gather(c)
        for c in range(n_total):
            b = c % NBUF
            _, out_h, _, ooff, do_norm = chunks[c]
            g_desc[c].wait()
            if c + PREF < n_total:
                if c + PREF - NBUF >= 0:
                    s_desc[c + PREF - NBUF].wait()
                start_gather(c + PREF)
            if do_norm:
                _normalize_chunk(rows_v.at[b], CHUNK, perms)
            s_desc[c] = pltpu.async_copy(rows_v.at[b],
                                         out_h.at[pl.ds(base + ooff, CHUNK)],
                                         ssem[b])
        for c in range(n_total - NBUF, n_total):
            if s_desc[c] is not None:
                s_desc[c].wait()

    return tuple(k(iph, ipr, ipt, inh, inr, int_, ent, rel))


def kernel(rel_pos_hs, rel_pos_rs, rel_pos_ts,
           rel_neg_hs, rel_neg_rs, rel_neg_ts,
           rv_ent_embeds, rel_embeds):
    def prep(ix):
        return ix.astype(jnp.int32).reshape(B // CHUNK, CHUNK)

    rel_padded = jnp.pad(rel_embeds, ((0, REL_PAD - NUM_REL), (0, 0)))
    return _sc_lookup(prep(rel_pos_hs), prep(rel_pos_rs), prep(rel_pos_ts),
                      prep(rel_neg_hs), prep(rel_neg_rs), prep(rel_neg_ts),
                      rv_ent_embeds, rel_padded)
